# baseline (speedup 1.0000x reference)
import jax
import jax.numpy as jnp
from jax import lax
from jax.experimental import pallas as pl
from jax.experimental.pallas import tpu as pltpu
from jax.experimental.pallas import tpu_sc as plsc

N = 50000
C = 64
E = 800000
XROWS = 50176
RPT = 3136
BLK = 128
NBLK = 16
CHUNK = NBLK * BLK
NGRP = 13
EPT = CHUNK * NGRP
EPAD = 32 * EPT
NBUF = 6
GD = 5
DUMP = 50168


def _sc_body(x2_hbm, src_hbm, tgt_hbm, s1_hbm, s2_hbm,
             gidx, sidx, lidx, rows, isem0, isem1, gsem, ssem, accum):
    c = lax.axis_index("c")
    s = lax.axis_index("s")
    tile = c * 16 + s

    for g_hbm, sc_hbm, out_hbm in ((src_hbm, tgt_hbm, s1_hbm),
                                   (tgt_hbm, src_hbm, s2_hbm)):
        pltpu.sync_copy(x2_hbm.at[pl.ds(c * XROWS + s * RPT, RPT)],
                        accum.at[pl.ds(s * RPT, RPT)])
        plsc.subcore_barrier()

        def group(g, carry):
            row0 = tile * (EPT // BLK) + g * NBLK
            d1 = pltpu.async_copy(g_hbm.at[pl.ds(row0, NBLK)], gidx, isem0)
            d2 = pltpu.async_copy(sc_hbm.at[pl.ds(row0, NBLK)], sidx, isem1)
            d1.wait()
            d2.wait()
            for b in range(NBLK):
                for j in range(BLK // 16):
                    v = sidx[b, pl.ds(j * 16, 16)]
                    lidx[b, pl.ds(j * 16, 16)] = jnp.where(v < N, v, DUMP)
            gd = [None] * NBLK
            sd = [None] * NBLK
            sdone = [False] * NBLK
            for b in range(min(GD, NBLK)):
                gd[b] = pltpu.async_copy(
                    x2_hbm.at[gidx.at[b]], rows.at[b % NBUF], gsem[b % NBUF])
            for b in range(NBLK):
                gd[b].wait()
                sd[b] = pltpu.async_copy(
                    rows.at[b % NBUF], accum.at[lidx.at[b]],
                    ssem[b % NBUF], add=True)
                nb = b + GD
                if nb < NBLK:
                    prev = nb - NBUF
                    if prev >= 0:
                        sd[prev].wait()
                        sdone[prev] = True
                    gd[nb] = pltpu.async_copy(
                        x2_hbm.at[gidx.at[nb]],
                        rows.at[nb % NBUF], gsem[nb % NBUF])
            for b in range(NBLK):
                if not sdone[b]:
                    sd[b].wait()
            return carry

        lax.fori_loop(0, NGRP, group, 0)
        plsc.subcore_barrier()
        pltpu.sync_copy(accum.at[pl.ds(s * RPT, RPT)],
                        out_hbm.at[pl.ds(c * XROWS + s * RPT, RPT)])
        plsc.subcore_barrier()


def _tc_body(s1_ref, s2_ref, n_ref, nt_ref, w_ref, o_ref):
    a1 = (s1_ref[0].astype(jnp.float32)
          + s1_ref[1].astype(jnp.float32)) * n_ref[...]
    a2 = (s2_ref[0].astype(jnp.float32)
          + s2_ref[1].astype(jnp.float32)) * nt_ref[...]
    a = jnp.concatenate([a1, a2], axis=1)
    o_ref[...] = jnp.dot(a, w_ref[...], preferred_element_type=jnp.float32)


def kernel(x, sources, targets, norm, norm_t, W_out, W_back):
    src = jnp.asarray(sources, jnp.int32)
    tgt = jnp.asarray(targets, jnp.int32)
    padv = N + (jnp.arange(EPAD - E, dtype=jnp.int32) % (XROWS - 8 - N))
    srcp = jnp.concatenate([src, padv]).reshape(EPAD // BLK, BLK)
    tgtp = jnp.concatenate([tgt, padv]).reshape(EPAD // BLK, BLK)
    x2 = jnp.zeros((2 * XROWS, C), jnp.bfloat16).at[:N].set(
        x.astype(jnp.bfloat16))

    mesh = plsc.VectorSubcoreMesh(core_axis_name="c", subcore_axis_name="s")
    s1, s2 = pl.kernel(
        _sc_body,
        out_type=(jax.ShapeDtypeStruct((2 * XROWS, C), jnp.bfloat16),
                  jax.ShapeDtypeStruct((2 * XROWS, C), jnp.bfloat16)),
        mesh=mesh,
        scratch_types=[
            pltpu.VMEM((NBLK, BLK), jnp.int32),
            pltpu.VMEM((NBLK, BLK), jnp.int32),
            pltpu.VMEM((NBLK, BLK), jnp.int32),
            pltpu.VMEM((NBUF, BLK, C), jnp.bfloat16),
            pltpu.SemaphoreType.DMA,
            pltpu.SemaphoreType.DMA,
            [pltpu.SemaphoreType.DMA] * NBUF,
            [pltpu.SemaphoreType.DMA] * NBUF,
            pltpu.VMEM_SHARED((XROWS, C), jnp.bfloat16),
        ],
        compiler_params=pltpu.CompilerParams(use_tc_tiling_on_sc=False),
    )(x2, srcp, tgtp)

    s1_3 = s1.reshape(2, XROWS, C)
    s2_3 = s2.reshape(2, XROWS, C)
    W_cat = jnp.concatenate([W_out, W_back], axis=0)

    out = pl.pallas_call(
        _tc_body,
        grid=(50,),
        in_specs=[
            pl.BlockSpec((2, 1000, C), lambda i: (0, i, 0)),
            pl.BlockSpec((2, 1000, C), lambda i: (0, i, 0)),
            pl.BlockSpec((1000, 1), lambda i: (i, 0)),
            pl.BlockSpec((1000, 1), lambda i: (i, 0)),
            pl.BlockSpec((2 * C, C), lambda i: (0, 0)),
        ],
        out_specs=pl.BlockSpec((1000, C), lambda i: (i, 0)),
        out_shape=jax.ShapeDtypeStruct((N, C), jnp.float32),
    )(s1_3, s2_3, norm, norm_t, W_cat)
    return out


# speedup vs baseline: 1.1773x; 1.1773x over previous
import jax
import jax.numpy as jnp
from jax import lax
from jax.experimental import pallas as pl
from jax.experimental.pallas import tpu as pltpu
from jax.experimental.pallas import tpu_sc as plsc

N = 50000
C = 64
E = 800000
XROWS = 50176
RPT = 3136
BLK = 128
NBLK = 16
EBLKS = E // BLK
BPT = EBLKS // 32
NGRP = BPT // NBLK
TAIL = BPT - NGRP * NBLK
NBUF = 6
GD = 5
DUMP = 50168


def _block_pipeline(x_hbm, g_hbm, sc_hbm, accum, gidx, sidx, lidx, rows,
                    isem0, isem1, gsem, ssem, eblk0, nblk):
    eoff = eblk0 * BLK
    d1 = pltpu.async_copy(g_hbm.at[pl.ds(eoff, nblk * BLK)],
                          gidx.at[pl.ds(0, nblk * BLK)], isem0)
    d2 = pltpu.async_copy(sc_hbm.at[pl.ds(eoff, nblk * BLK)],
                          sidx.at[pl.ds(0, nblk * BLK)], isem1)
    d1.wait()
    d2.wait()
    for b in range(nblk):
        for j in range(BLK // 16):
            v = sidx[pl.ds(b * BLK + j * 16, 16)]
            lidx[b, pl.ds(j * 16, 16)] = jnp.where(v < N, v, DUMP)
    gd = [None] * nblk
    sd = [None] * nblk
    sdone = [False] * nblk
    for b in range(min(GD, nblk)):
        gd[b] = pltpu.async_copy(
            x_hbm.at[gidx.at[pl.ds(b * BLK, BLK)]],
            rows.at[b % NBUF], gsem[b % NBUF])
    for b in range(nblk):
        gd[b].wait()
        sd[b] = pltpu.async_copy(
            rows.at[b % NBUF], accum.at[lidx.at[b]],
            ssem[b % NBUF], add=True)
        nb = b + GD
        if nb < nblk:
            prev = nb - NBUF
            if prev >= 0:
                sd[prev].wait()
                sdone[prev] = True
            gd[nb] = pltpu.async_copy(
                x_hbm.at[gidx.at[pl.ds(nb * BLK, BLK)]],
                rows.at[nb % NBUF], gsem[nb % NBUF])
    for b in range(nblk):
        if not sdone[b]:
            sd[b].wait()


def _sc_body(x_hbm, zer_hbm, src_hbm, tgt_hbm, s1_hbm, s2_hbm,
             gidx, sidx, lidx, rows, isem0, isem1, gsem, ssem, accum):
    c = lax.axis_index("c")
    s = lax.axis_index("s")
    tile = c * 16 + s
    row0 = tile * BPT + jnp.minimum(tile, 10)

    for g_hbm, sc_hbm, out_hbm in ((src_hbm, tgt_hbm, s1_hbm),
                                   (tgt_hbm, src_hbm, s2_hbm)):
        @pl.when(c == 0)
        def _():
            @pl.when(s < 15)
            def _():
                pltpu.sync_copy(x_hbm.at[pl.ds(s * RPT, RPT)],
                                accum.at[pl.ds(s * RPT, RPT)])
            @pl.when(s == 15)
            def _():
                pltpu.sync_copy(x_hbm.at[pl.ds(15 * RPT, N - 15 * RPT)],
                                accum.at[pl.ds(15 * RPT, N - 15 * RPT)])
        @pl.when(c == 1)
        def _():
            pltpu.sync_copy(zer_hbm, accum.at[pl.ds(s * RPT, RPT)])
        plsc.subcore_barrier()

        args = (x_hbm, g_hbm, sc_hbm, accum, gidx, sidx, lidx, rows,
                isem0, isem1, gsem, ssem)

        def group(g, carry):
            _block_pipeline(*args, row0 + g * NBLK, NBLK)
            return carry

        lax.fori_loop(0, NGRP, group, 0)
        _block_pipeline(*args, row0 + NGRP * NBLK, TAIL)

        @pl.when(tile < 10)
        def _():
            _block_pipeline(*args, row0 + BPT, 1)

        plsc.subcore_barrier()
        pltpu.sync_copy(accum.at[pl.ds(s * RPT, RPT)],
                        out_hbm.at[c, pl.ds(s * RPT, RPT)])
        plsc.subcore_barrier()


def _tc_body(s1_ref, s2_ref, n_ref, nt_ref, w_ref, o_ref):
    a1 = (s1_ref[0].astype(jnp.float32)
          + s1_ref[1].astype(jnp.float32)) * n_ref[...]
    a2 = (s2_ref[0].astype(jnp.float32)
          + s2_ref[1].astype(jnp.float32)) * nt_ref[...]
    a = jnp.concatenate([a1, a2], axis=1)
    o_ref[...] = jnp.dot(a, w_ref[...], preferred_element_type=jnp.float32)


def kernel(x, sources, targets, norm, norm_t, W_out, W_back):
    src = jnp.asarray(sources, jnp.int32)
    tgt = jnp.asarray(targets, jnp.int32)
    x_bf = x.astype(jnp.bfloat16)
    zer = jnp.zeros((RPT, C), jnp.bfloat16)

    mesh = plsc.VectorSubcoreMesh(core_axis_name="c", subcore_axis_name="s")
    s1, s2 = pl.kernel(
        _sc_body,
        out_type=(jax.ShapeDtypeStruct((2, XROWS, C), jnp.bfloat16),
                  jax.ShapeDtypeStruct((2, XROWS, C), jnp.bfloat16)),
        mesh=mesh,
        scratch_types=[
            pltpu.VMEM((NBLK * BLK,), jnp.int32),
            pltpu.VMEM((NBLK * BLK,), jnp.int32),
            pltpu.VMEM((NBLK, BLK), jnp.int32),
            pltpu.VMEM((NBUF, BLK, C), jnp.bfloat16),
            pltpu.SemaphoreType.DMA,
            pltpu.SemaphoreType.DMA,
            [pltpu.SemaphoreType.DMA] * NBUF,
            [pltpu.SemaphoreType.DMA] * NBUF,
            pltpu.VMEM_SHARED((XROWS, C), jnp.bfloat16),
        ],
        compiler_params=pltpu.CompilerParams(use_tc_tiling_on_sc=False),
    )(x_bf, zer, src, tgt)

    W_cat = jnp.concatenate([W_out, W_back], axis=0)

    out = pl.pallas_call(
        _tc_body,
        grid=(50,),
        in_specs=[
            pl.BlockSpec((2, 1000, C), lambda i: (0, i, 0)),
            pl.BlockSpec((2, 1000, C), lambda i: (0, i, 0)),
            pl.BlockSpec((1000, 1), lambda i: (i, 0)),
            pl.BlockSpec((1000, 1), lambda i: (i, 0)),
            pl.BlockSpec((2 * C, C), lambda i: (0, 0)),
        ],
        out_specs=pl.BlockSpec((1000, C), lambda i: (i, 0)),
        out_shape=jax.ShapeDtypeStruct((N, C), jnp.float32),
    )(s1, s2, norm, norm_t, W_cat)
    return out


# speedup vs baseline: 1.2468x; 1.0590x over previous
import jax
import jax.numpy as jnp
from jax import lax
from jax.experimental import pallas as pl
from jax.experimental.pallas import tpu as pltpu
from jax.experimental.pallas import tpu_sc as plsc

N = 50000
C = 64
E = 800000
XROWS = 50176
RPT = 3136
BLK = 128
NBLK = 16
EBLKS = E // BLK
BPT = EBLKS // 32
NGRP = BPT // NBLK
TAIL = BPT - NGRP * NBLK
NBUF = 6
GD = 5
DUMP = 50168


def _block_pipeline(x_hbm, g_hbm, sc_hbm, accum, gidx, sidx, lidx, rows,
                    isem0, isem1, gsem, ssem, eblk0, nblk):
    eoff = eblk0 * BLK
    d1 = pltpu.async_copy(g_hbm.at[pl.ds(eoff, nblk * BLK)],
                          gidx.at[pl.ds(0, nblk * BLK)], isem0)
    d2 = pltpu.async_copy(sc_hbm.at[pl.ds(eoff, nblk * BLK)],
                          sidx.at[pl.ds(0, nblk * BLK)], isem1)
    d1.wait()
    d2.wait()
    for b in range(nblk):
        for j in range(BLK // 16):
            v = sidx[pl.ds(b * BLK + j * 16, 16)]
            lidx[b, pl.ds(j * 16, 16)] = jnp.where(v < N, v, DUMP)
    gd = [None] * nblk
    sd = [None] * nblk
    sdone = [False] * nblk
    for b in range(min(GD, nblk)):
        gd[b] = pltpu.async_copy(
            x_hbm.at[gidx.at[pl.ds(b * BLK, BLK)]],
            rows.at[b % NBUF], gsem[b % NBUF])
    for b in range(nblk):
        gd[b].wait()
        sd[b] = pltpu.async_copy(
            rows.at[b % NBUF], accum.at[lidx.at[b]],
            ssem[b % NBUF], add=True)
        nb = b + GD
        if nb < nblk:
            prev = nb - NBUF
            if prev >= 0:
                sd[prev].wait()
                sdone[prev] = True
            gd[nb] = pltpu.async_copy(
                x_hbm.at[gidx.at[pl.ds(nb * BLK, BLK)]],
                rows.at[nb % NBUF], gsem[nb % NBUF])
    for b in range(nblk):
        if not sdone[b]:
            sd[b].wait()


def _sc_body(x_hbm, zer_hbm, g_hbm, sc_hbm, out_hbm,
             gidx, sidx, lidx, rows, isem0, isem1, gsem, ssem, accum):
    c = lax.axis_index("c")
    s = lax.axis_index("s")
    tile = c * 16 + s
    row0 = tile * BPT + jnp.minimum(tile, 10)

    @pl.when(c == 0)
    def _():
        @pl.when(s < 15)
        def _():
            pltpu.sync_copy(x_hbm.at[pl.ds(s * RPT, RPT)],
                            accum.at[pl.ds(s * RPT, RPT)])
        @pl.when(s == 15)
        def _():
            pltpu.sync_copy(x_hbm.at[pl.ds(15 * RPT, N - 15 * RPT)],
                            accum.at[pl.ds(15 * RPT, N - 15 * RPT)])
    @pl.when(c == 1)
    def _():
        pltpu.sync_copy(zer_hbm, accum.at[pl.ds(s * RPT, RPT)])
    plsc.subcore_barrier()

    args = (x_hbm, g_hbm, sc_hbm, accum, gidx, sidx, lidx, rows,
            isem0, isem1, gsem, ssem)

    def group(g, carry):
        _block_pipeline(*args, row0 + g * NBLK, NBLK)
        return carry

    lax.fori_loop(0, NGRP, group, 0)
    _block_pipeline(*args, row0 + NGRP * NBLK, TAIL)

    @pl.when(tile < 10)
    def _():
        _block_pipeline(*args, row0 + BPT, 1)

    plsc.subcore_barrier()
    pltpu.sync_copy(accum.at[pl.ds(s * RPT, RPT)],
                    out_hbm.at[c, pl.ds(s * RPT, RPT)])


def _tc_body1(s_ref, n_ref, w_ref, o_ref):
    a = (s_ref[0].astype(jnp.float32)
         + s_ref[1].astype(jnp.float32)) * n_ref[...]
    o_ref[...] = jnp.dot(a, w_ref[...], preferred_element_type=jnp.float32)


def _tc_body2(s_ref, n_ref, w_ref, p_ref, o_ref):
    a = (s_ref[0].astype(jnp.float32)
         + s_ref[1].astype(jnp.float32)) * n_ref[...]
    o_ref[...] = p_ref[...] + jnp.dot(
        a, w_ref[...], preferred_element_type=jnp.float32)


def kernel(x, sources, targets, norm, norm_t, W_out, W_back):
    src = jnp.asarray(sources, jnp.int32)
    tgt = jnp.asarray(targets, jnp.int32)
    x_bf = x.astype(jnp.bfloat16)
    zer = jnp.zeros((RPT, C), jnp.bfloat16)

    mesh = plsc.VectorSubcoreMesh(core_axis_name="c", subcore_axis_name="s")
    sc_call = pl.kernel(
        _sc_body,
        out_type=jax.ShapeDtypeStruct((2, XROWS, C), jnp.bfloat16),
        mesh=mesh,
        scratch_types=[
            pltpu.VMEM((NBLK * BLK,), jnp.int32),
            pltpu.VMEM((NBLK * BLK,), jnp.int32),
            pltpu.VMEM((NBLK, BLK), jnp.int32),
            pltpu.VMEM((NBUF, BLK, C), jnp.bfloat16),
            pltpu.SemaphoreType.DMA,
            pltpu.SemaphoreType.DMA,
            [pltpu.SemaphoreType.DMA] * NBUF,
            [pltpu.SemaphoreType.DMA] * NBUF,
            pltpu.VMEM_SHARED((XROWS, C), jnp.bfloat16),
        ],
        compiler_params=pltpu.CompilerParams(use_tc_tiling_on_sc=False),
    )
    s1 = sc_call(x_bf, zer, src, tgt)
    s2 = sc_call(x_bf, zer, tgt, src)

    spec_s = pl.BlockSpec((2, 1000, C), lambda i: (0, i, 0))
    spec_n = pl.BlockSpec((1000, 1), lambda i: (i, 0))
    spec_w = pl.BlockSpec((C, C), lambda i: (0, 0))
    spec_o = pl.BlockSpec((1000, C), lambda i: (i, 0))
    out_sds = jax.ShapeDtypeStruct((N, C), jnp.float32)

    part = pl.pallas_call(
        _tc_body1,
        grid=(50,),
        in_specs=[spec_s, spec_n, spec_w],
        out_specs=spec_o,
        out_shape=out_sds,
    )(s1, norm, W_out)

    out = pl.pallas_call(
        _tc_body2,
        grid=(50,),
        in_specs=[spec_s, spec_n, spec_w, spec_o],
        out_specs=spec_o,
        out_shape=out_sds,
    )(s2, norm_t, W_back, part)
    return out
